# baseline (speedup 1.0000x reference)
import jax
import jax.numpy as jnp
from jax.experimental import pallas as pl
from jax.experimental.pallas import tpu as pltpu

_ROWS = 100000
_BR = 2000


def _body(q_ref, r_ref, qo_ref, ro_ref):
    q = q_ref[...]
    qo_ref[...] = jnp.round(jnp.clip(q, 0.0, 1.0) * 255.0) * (1.0 / 255.0)
    r = r_ref[...]
    m = jnp.max(r, axis=-1, keepdims=True)
    e = jnp.exp(r - m)
    ro_ref[...] = e / jnp.sum(e, axis=-1, keepdims=True)


def kernel(query, response):
    n = query.shape[0]
    grid = (n // _BR,)
    qo, ro = pl.pallas_call(
        _body,
        grid=grid,
        in_specs=[
            pl.BlockSpec((_BR, query.shape[1]), lambda i: (i, 0)),
            pl.BlockSpec((_BR, response.shape[1]), lambda i: (i, 0)),
        ],
        out_specs=[
            pl.BlockSpec((_BR, query.shape[1]), lambda i: (i, 0)),
            pl.BlockSpec((_BR, response.shape[1]), lambda i: (i, 0)),
        ],
        out_shape=[
            jax.ShapeDtypeStruct(query.shape, query.dtype),
            jax.ShapeDtypeStruct(response.shape, response.dtype),
        ],
        compiler_params=pltpu.CompilerParams(
            dimension_semantics=("parallel",),
        ),
    )(query, response)
    return (qo, ro)
